# baseline (speedup 1.0000x reference)
import functools

import jax
import jax.numpy as jnp
from jax import lax
from jax.experimental import pallas as pl
from jax.experimental.pallas import tpu as pltpu
from jax.experimental.pallas import tpu_sc as plsc

N = 10000
E = 160000
B = 64
DF = 256
DE = 16
GIN = 64
HID = 512
OUT = 256
NHID = 3

NC = 2
NS = 16
NW = NC * NS
CH = 125
NCHUNK = E // CH
TILE_C = NCHUNK // NW
EPT = TILE_C * CH
NROW = 10240
RPT = NROW // NS


def _sc_edge_scatter(ei3d, attr):
    mesh = plsc.VectorSubcoreMesh(core_axis_name="c", subcore_axis_name="s")

    @functools.partial(
        pl.kernel,
        out_type=(
            jax.ShapeDtypeStruct((NC, NROW, DE), jnp.float32),
            jax.ShapeDtypeStruct((NC, NROW, DE), jnp.float32),
        ),
        mesh=mesh,
        compiler_params=pltpu.CompilerParams(use_tc_tiling_on_sc=False),
        scratch_types=[
            pltpu.VMEM((TILE_C, CH), jnp.int32),
            pltpu.VMEM((EPT, DE), jnp.float32),
            pltpu.VMEM((RPT, DE), jnp.float32),
            pltpu.VMEM((CH, DE), jnp.float32),
            pltpu.VMEM_SHARED((NROW, DE), jnp.float32),
            pltpu.VMEM_SHARED((NROW, DE), jnp.float32),
            pltpu.SemaphoreType.DMA,
            pltpu.SemaphoreType.DMA,
        ],
    )
    def k(dst_hbm, attr_hbm, val_out, cnt_out,
          idx_all, attr_all, zrow, ones_rows,
          val_sh, cnt_sh, sem_v, sem_c):
        c = lax.axis_index("c")
        s = lax.axis_index("s")
        wid = c * NS + s

        zeros16 = jnp.zeros((16,), jnp.float32)
        lane = lax.iota(jnp.int32, 16)
        onehot = jnp.where(lane == 0, 1.0, 0.0).astype(jnp.float32)

        h_idx = pltpu.async_copy(dst_hbm.at[1, pl.ds(wid * TILE_C, TILE_C)],
                                 idx_all, sem_v)
        h_attr = pltpu.async_copy(attr_hbm.at[pl.ds(wid * EPT, EPT)],
                                  attr_all, sem_c)

        def _zero_zrow(i, carry):
            zrow[i, :] = zeros16
            return carry
        lax.fori_loop(0, RPT, _zero_zrow, 0)

        def _fill_ones(i, carry):
            ones_rows[i, :] = onehot
            return carry
        lax.fori_loop(0, CH, _fill_ones, 0)

        pltpu.sync_copy(zrow, val_sh.at[pl.ds(s * RPT, RPT)])
        pltpu.sync_copy(zrow, cnt_sh.at[pl.ds(s * RPT, RPT)])

        plsc.subcore_barrier()
        h_idx.wait()
        h_attr.wait()

        WAVE = 20

        def _wave(w, carry):
            hs = []
            for j in range(WAVE):
                ci = w * WAVE + j
                hs.append(pltpu.async_copy(
                    attr_all.at[pl.ds(ci * CH, CH)],
                    val_sh.at[idx_all.at[ci]], sem_v, add=True))
                hs.append(pltpu.async_copy(
                    ones_rows, cnt_sh.at[idx_all.at[ci]], sem_c, add=True))
            for h in hs:
                h.wait()
            return carry
        lax.fori_loop(0, TILE_C // WAVE, _wave, 0)

        plsc.subcore_barrier()

        h1 = pltpu.async_copy(val_sh.at[pl.ds(s * RPT, RPT)],
                              val_out.at[c, pl.ds(s * RPT, RPT)], sem_v)
        h2 = pltpu.async_copy(cnt_sh.at[pl.ds(s * RPT, RPT)],
                              cnt_out.at[c, pl.ds(s * RPT, RPT)], sem_c)
        h1.wait()
        h2.wait()

    return k(ei3d, attr)


NBLK = 5
CHN = N // NBLK
NRV = NROW * DE // 128
NPK = 128 // DE


def _elu(v):
    return jnp.where(v > 0, v, jnp.exp(v) - 1.0)


def _dot_nt(a, w):
    return lax.dot_general(a, w, (((1,), (1,)), ((), ())),
                           preferred_element_type=jnp.float32)


def _tc1_body(x_ref, b_ref, xs_ref, cs_ref, accx, accc):
    i = pl.program_id(0)

    @pl.when(i == 0)
    def _():
        accx[...] = jnp.zeros_like(accx)
        accc[...] = jnp.zeros_like(accc)

    bblk = b_ref[0]
    iota = lax.broadcasted_iota(jnp.int32, (B, CHN), 0)
    mask = (bblk == iota).astype(jnp.float32)
    accx[...] += jnp.dot(mask, x_ref[...], preferred_element_type=jnp.float32)
    accc[...] += jnp.sum(mask, axis=1, keepdims=True)

    @pl.when(i == NBLK - 1)
    def _():
        xs_ref[...] = accx[...]
        cs_ref[...] = jnp.broadcast_to(accc[...], (B, 128))


def _tc1(x, batch3d):
    return pl.pallas_call(
        _tc1_body,
        grid=(NBLK,),
        in_specs=[
            pl.BlockSpec((CHN, DF), lambda i: (i, 0)),
            pl.BlockSpec((1, 1, CHN), lambda i: (i, 0, 0)),
        ],
        out_specs=[pl.BlockSpec((B, DF), lambda i: (0, 0)),
                   pl.BlockSpec((B, 128), lambda i: (0, 0))],
        out_shape=[jax.ShapeDtypeStruct((B, DF), jnp.float32),
                   jax.ShapeDtypeStruct((B, 128), jnp.float32)],
        scratch_shapes=[
            pltpu.VMEM((B, DF), jnp.float32),
            pltpu.VMEM((B, 1), jnp.float32),
        ],
    )(x, batch3d)


def _tc2_body(b8_ref, u_ref, val_ref, cnt_ref, xs_ref, cs_ref, win_ref,
              bin_ref, wh_ref, bh_ref, wo_ref, g_ref, beta_ref, out_ref):
    c128 = cnt_ref[0] + cnt_ref[1]
    kk = lax.broadcasted_iota(jnp.int32, (128, 128), 0)
    ll = lax.broadcasted_iota(jnp.int32, (128, 128), 1)
    sel = (kk == (ll // DE) * DE).astype(jnp.float32)
    cb = jnp.dot(c128, sel, preferred_element_type=jnp.float32)
    em128 = (val_ref[0] + val_ref[1]) / jnp.maximum(cb, 1.0)
    iota8 = lax.broadcasted_iota(jnp.int32, (B, NRV), 0)
    ae = jnp.zeros((B, DE), jnp.float32)
    for a in range(NPK):
        mask_a = (b8_ref[a:a + 1, :] == iota8).astype(jnp.float32)
        ae = ae + jnp.dot(mask_a, em128[:, a * DE:(a + 1) * DE],
                          preferred_element_type=jnp.float32)

    cnt = jnp.maximum(cs_ref[:, 0:1], 1.0)
    node_mean = xs_ref[...] / cnt
    edge_mean = ae / cnt
    feat = jnp.concatenate([u_ref[...], node_mean, edge_mean], axis=1)
    h = _elu(_dot_nt(feat, win_ref[...]) + bin_ref[...])
    skip = h
    for l in range(NHID):
        h = _elu(_dot_nt(h, wh_ref[l]) + bh_ref[l])
    h = h + skip
    z = _dot_nt(h, wo_ref[...])
    mu = jnp.mean(z, axis=-1, keepdims=True)
    var = jnp.mean((z - mu) ** 2, axis=-1, keepdims=True)
    z = (z - mu) / jnp.sqrt(var + 1e-5) * g_ref[...] + beta_ref[...]
    out_ref[...] = _elu(z)


def _tc2(batch8, u, val, cnt, xs, cs, W_in, bin2, W_hid, bh3, W_out,
         g2, beta2):
    return pl.pallas_call(
        _tc2_body,
        out_shape=jax.ShapeDtypeStruct((B, OUT), jnp.float32),
    )(batch8, u, val, cnt, xs, cs, W_in, bin2, W_hid, bh3, W_out, g2, beta2)


def kernel(x, edge_index, edge_attr, u, batch, W_in, b_in, W_hid, b_hid,
           W_out, gamma, beta):
    ei3d = edge_index.reshape(2, NCHUNK, CH)
    val, cnt = _sc_edge_scatter(ei3d, edge_attr)
    val = val.reshape(NC, NRV, 128)
    cnt = cnt.reshape(NC, NRV, 128)

    batch3d = batch.reshape(NBLK, 1, CHN)
    batch8 = jnp.pad(batch, (0, NROW - N), constant_values=B)
    batch8 = batch8.reshape(NRV, NPK).T
    xs, cs = _tc1(x, batch3d)
    return _tc2(batch8, u, val, cnt, xs, cs, W_in, b_in[None, :], W_hid,
                b_hid[:, None, :], W_out, gamma[None, :], beta[None, :])
